# baseline (speedup 1.0000x reference)
import functools

import jax
import jax.numpy as jnp
from jax import lax
from jax.experimental import pallas as pl
from jax.experimental.pallas import tpu as pltpu
from jax.experimental.pallas import tpu_sc as plsc

_NBUF = 3
_LEAD = 3
_RING = 6


def _make_gather_t(V, D, B0, B1):
    info = plsc.get_sparse_core_info()
    NC, NS, L = info.num_cores, info.num_subcores, info.num_lanes
    assert NC * NS == D and B0 % L == 0
    n_groups = B0 // L

    mesh = plsc.VectorSubcoreMesh(core_axis_name="c", subcore_axis_name="s")

    @functools.partial(
        pl.kernel,
        mesh=mesh,
        out_type=jax.ShapeDtypeStruct((B1, D, B0), jnp.float32),
        scratch_types=[
            pltpu.VMEM((V,), jnp.float32),
            [pltpu.VMEM((B0,), jnp.int32) for _ in range(_NBUF)],
            [pltpu.VMEM((B0,), jnp.float32) for _ in range(_NBUF)],
            pltpu.VMEM_SHARED((_RING * B0,), jnp.int32),
            [pltpu.SemaphoreType.DMA for _ in range(_NBUF)],
            [pltpu.SemaphoreType.DMA for _ in range(_NBUF)],
            pltpu.SemaphoreType.DMA,
        ],
        compiler_params=pltpu.CompilerParams(use_tc_tiling_on_sc=True,
                                             needs_layout_passes=False),
    )
    def gather_kernel(table_hbm, ct_hbm, out_hbm, row_v, idx_bufs, out_bufs,
                      ct_ring, idx_sems, out_sems, stage_sem):
        s = lax.axis_index("s")
        k = lax.axis_index("c") * NS + s

        def ring_slot(j):
            return ct_ring.at[pl.ds((j % _RING) * B0, B0)]

        def start_stage(j):
            pltpu.async_copy(ct_hbm.at[j], ring_slot(j), stage_sem)

        def wait_stage(j):
            pltpu.make_async_copy(ct_hbm.at[j], ring_slot(j),
                                  stage_sem).wait()

        def start_idx(j, b):
            pltpu.async_copy(ring_slot(j), idx_bufs[b], idx_sems[b])

        def wait_idx(j, b):
            pltpu.make_async_copy(ring_slot(j), idx_bufs[b],
                                  idx_sems[b]).wait()

        def start_out(j, b):
            pltpu.async_copy(out_bufs[b], out_hbm.at[j, k], out_sems[b])

        def wait_out(j, b):
            pltpu.make_async_copy(out_bufs[b], out_hbm.at[j, k],
                                  out_sems[b]).wait()

        def gather_row(b):
            idx_v, out_v = idx_bufs[b], out_bufs[b]

            @plsc.parallel_loop(0, n_groups, unroll=32)
            def _(g):
                idx16 = idx_v[pl.ds(g * L, L)]
                out_v[pl.ds(g * L, L)] = plsc.load_gather(row_v, [idx16])

        @pl.when(s == 0)
        def _():
            for j in range(2 * _LEAD - 1):
                start_stage(j)

        pltpu.sync_copy(table_hbm.at[k], row_v)

        @pl.when(s == 0)
        def _():
            for j in range(_NBUF - 1):
                wait_stage(j)

        plsc.subcore_barrier()
        for j in range(_NBUF - 1):
            start_idx(j, j)

        def body(i, carry):
            for p in range(2 * _NBUF):
                j = 2 * _NBUF * i + p
                b = p % _NBUF

                @pl.when((s == 0) & (j + _LEAD - 1 < B1))
                def _():
                    wait_stage(j + _LEAD - 1)

                plsc.subcore_barrier()

                @pl.when((s == 0) & (j + 2 * _LEAD - 1 < B1))
                def _():
                    start_stage(j + 2 * _LEAD - 1)

                @pl.when(j + _NBUF - 1 < B1)
                def _():
                    start_idx(j + _NBUF - 1, (p + _NBUF - 1) % _NBUF)

                wait_idx(j, b)

                if p >= _NBUF:
                    wait_out(j - _NBUF, b)
                else:
                    @pl.when(i > 0)
                    def _():
                        wait_out(j - _NBUF, b)

                gather_row(b)
                start_out(j, b)
            return carry

        n_iters = B1 // (2 * _NBUF)
        lax.fori_loop(0, n_iters, body, 0)
        rem = B1 - n_iters * 2 * _NBUF
        for p in range(rem):
            j = n_iters * 2 * _NBUF + p
            b = j % _NBUF

            @pl.when((s == 0) & (j + _LEAD - 1 < B1))
            def _():
                wait_stage(j + _LEAD - 1)

            plsc.subcore_barrier()

            @pl.when(j + _NBUF - 1 < B1)
            def _():
                start_idx(j + _NBUF - 1, (j + _NBUF - 1) % _NBUF)

            wait_idx(j, b)
            wait_out(j - _NBUF, b)
            gather_row(b)
            start_out(j, b)
        for j in range(B1 - _NBUF, B1):
            wait_out(j, j % _NBUF)

    return gather_kernel


def kernel(cell_types, table):
    B0, B1 = cell_types.shape
    V, D = table.shape
    ct_t = jnp.transpose(cell_types.astype(jnp.int32))
    table_t = jnp.transpose(table)
    out_t = _make_gather_t(V, D, B0, B1)(table_t, ct_t)
    return jnp.transpose(out_t, (2, 0, 1))


# speedup vs baseline: 1.0031x; 1.0031x over previous
import functools

import jax
import jax.numpy as jnp
from jax import lax
from jax.experimental import pallas as pl
from jax.experimental.pallas import tpu as pltpu
from jax.experimental.pallas import tpu_sc as plsc

_NBUF = 3
_LEAD = 3
_RING = 6


def _make_gather_t(V, D, B0, B1):
    info = plsc.get_sparse_core_info()
    NC, NS, L = info.num_cores, info.num_subcores, info.num_lanes
    assert NC * NS == D and B0 % L == 0
    n_groups = B0 // L

    mesh = plsc.VectorSubcoreMesh(core_axis_name="c", subcore_axis_name="s")

    @functools.partial(
        pl.kernel,
        mesh=mesh,
        out_type=jax.ShapeDtypeStruct((B1, D, B0), jnp.float32),
        scratch_types=[
            pltpu.VMEM((V,), jnp.float32),
            [pltpu.VMEM((B0,), jnp.int32) for _ in range(_NBUF)],
            [pltpu.VMEM((B0,), jnp.float32) for _ in range(_NBUF)],
            pltpu.VMEM_SHARED((_RING * B0,), jnp.int32),
            [pltpu.SemaphoreType.DMA for _ in range(_NBUF)],
            [pltpu.SemaphoreType.DMA for _ in range(_NBUF)],
            pltpu.SemaphoreType.DMA,
        ],
        compiler_params=pltpu.CompilerParams(use_tc_tiling_on_sc=True,
                                             needs_layout_passes=False),
    )
    def gather_kernel(table_hbm, ct_hbm, out_hbm, row_v, idx_bufs, out_bufs,
                      ct_ring, idx_sems, out_sems, stage_sem):
        s = lax.axis_index("s")
        k = lax.axis_index("c") * NS + s

        def ring_slot(j):
            return ct_ring.at[pl.ds((j % _RING) * B0, B0)]

        def start_stage(j):
            pltpu.async_copy(ct_hbm.at[j], ring_slot(j), stage_sem)

        def wait_stage(j):
            pltpu.make_async_copy(ct_hbm.at[j], ring_slot(j),
                                  stage_sem).wait()

        def start_idx(j, b):
            pltpu.async_copy(ring_slot(j), idx_bufs[b], idx_sems[b])

        def wait_idx(j, b):
            pltpu.make_async_copy(ring_slot(j), idx_bufs[b],
                                  idx_sems[b]).wait()

        def start_out(j, b):
            pltpu.async_copy(out_bufs[b], out_hbm.at[j, k], out_sems[b])

        def wait_out(j, b):
            pltpu.make_async_copy(out_bufs[b], out_hbm.at[j, k],
                                  out_sems[b]).wait()

        def gather_row(b):
            idx_v, out_v = idx_bufs[b], out_bufs[b]

            @plsc.parallel_loop(0, n_groups, unroll=16)
            def _(g):
                idx16 = idx_v[pl.ds(g * L, L)]
                out_v[pl.ds(g * L, L)] = plsc.load_gather(row_v, [idx16])

        @pl.when(s == 0)
        def _():
            for j in range(2 * _LEAD - 1):
                start_stage(j)

        pltpu.sync_copy(table_hbm.at[k], row_v)

        @pl.when(s == 0)
        def _():
            for j in range(_NBUF - 1):
                wait_stage(j)

        plsc.subcore_barrier()
        for j in range(_NBUF - 1):
            start_idx(j, j)

        def body(i, carry):
            for p in range(2 * _NBUF):
                j = 2 * _NBUF * i + p
                b = p % _NBUF

                @pl.when((s == 0) & (j + _LEAD - 1 < B1))
                def _():
                    wait_stage(j + _LEAD - 1)

                plsc.subcore_barrier()

                @pl.when((s == 0) & (j + 2 * _LEAD - 1 < B1))
                def _():
                    start_stage(j + 2 * _LEAD - 1)

                @pl.when(j + _NBUF - 1 < B1)
                def _():
                    start_idx(j + _NBUF - 1, (p + _NBUF - 1) % _NBUF)

                wait_idx(j, b)

                if p >= _NBUF:
                    wait_out(j - _NBUF, b)
                else:
                    @pl.when(i > 0)
                    def _():
                        wait_out(j - _NBUF, b)

                gather_row(b)
                start_out(j, b)
            return carry

        n_iters = B1 // (2 * _NBUF)
        lax.fori_loop(0, n_iters, body, 0)
        rem = B1 - n_iters * 2 * _NBUF
        for p in range(rem):
            j = n_iters * 2 * _NBUF + p
            b = j % _NBUF

            @pl.when((s == 0) & (j + _LEAD - 1 < B1))
            def _():
                wait_stage(j + _LEAD - 1)

            plsc.subcore_barrier()

            @pl.when(j + _NBUF - 1 < B1)
            def _():
                start_idx(j + _NBUF - 1, (j + _NBUF - 1) % _NBUF)

            wait_idx(j, b)
            wait_out(j - _NBUF, b)
            gather_row(b)
            start_out(j, b)
        for j in range(B1 - _NBUF, B1):
            wait_out(j, j % _NBUF)

    return gather_kernel


def kernel(cell_types, table):
    B0, B1 = cell_types.shape
    V, D = table.shape
    ct_t = jnp.transpose(cell_types.astype(jnp.int32))
    table_t = jnp.transpose(table)
    out_t = _make_gather_t(V, D, B0, B1)(table_t, ct_t)
    return jnp.transpose(out_t, (2, 0, 1))
